# baseline (speedup 1.0000x reference)
import functools

import jax
import jax.numpy as jnp
from jax.experimental import pallas as pl
from jax.experimental.pallas import tpu as pltpu

_DECAY = 0.1
_EPS = 1e-8
_LN_EPS = 1e-5


def _tc_body(h_ref, lens_ref, scale_ref, bias_ref, out_ref, *, SEG, T, D):
    lens_f = lens_ref[...].astype(jnp.float32)
    col = jax.lax.broadcasted_iota(jnp.int32, (SEG, SEG * T), 1)
    srow = jax.lax.broadcasted_iota(jnp.int32, (SEG, SEG * T), 0)
    t = col % T
    sp = col // T
    w_base = jnp.exp(-_DECAY * ((T - 1) - t).astype(jnp.float32))
    valid = (t.astype(jnp.float32) < lens_f) & (sp == srow)
    wbd = jnp.where(valid, w_base, 0.0)
    e = jax.lax.dot_general(
        wbd, h_ref[...], (((1,), (0,)), ((), ())), preferred_element_type=jnp.float32
    )
    r = jnp.exp(jnp.float32(_DECAY))
    wsum = jnp.exp(-_DECAY * (T - 1)) * (jnp.exp(_DECAY * lens_f) - 1.0) / (r - 1.0)
    wsum = jnp.maximum(wsum, _EPS)
    e = e / wsum
    mu = jnp.mean(e, axis=1, keepdims=True)
    var = jnp.mean((e - mu) ** 2, axis=1, keepdims=True)
    e_ln = (e - mu) * jax.lax.rsqrt(var + _LN_EPS) * scale_ref[...] + bias_ref[...]
    out_ref[...] = jnp.where(lens_f >= 1.0, e_ln, e)


def kernel(H, valid_lens, ln_scale, ln_bias):
    B, F, T, D = H.shape
    S = B * F
    SEG = 16
    grid = (S // SEG,)
    H2 = H.reshape(S * T, D)
    lens2 = valid_lens.reshape(S, 1).astype(jnp.int32)
    scale2 = ln_scale.reshape(1, D)
    bias2 = ln_bias.reshape(1, D)

    out = pl.pallas_call(
        functools.partial(_tc_body, SEG=SEG, T=T, D=D),
        grid=grid,
        in_specs=[
            pl.BlockSpec((SEG * T, D), lambda i: (i, 0)),
            pl.BlockSpec((SEG, 1), lambda i: (i, 0)),
            pl.BlockSpec((1, D), lambda i: (0, 0)),
            pl.BlockSpec((1, D), lambda i: (0, 0)),
        ],
        out_specs=pl.BlockSpec((SEG, D), lambda i: (i, 0)),
        out_shape=jax.ShapeDtypeStruct((S, D), jnp.float32),
    )(H2, lens2, scale2, bias2)
    return out.reshape(B, F, D)


# speedup vs baseline: 1.6017x; 1.6017x over previous
import functools

import jax
import jax.numpy as jnp
from jax.experimental import pallas as pl

_DECAY = 0.1
_EPS = 1e-8
_LN_EPS = 1e-5


def _tc_body(h_ref, lens_ref, scale_ref, bias_ref, out_ref, *, SEG, T, D):
    lens_f = lens_ref[...].astype(jnp.float32)
    t_idx = jax.lax.broadcasted_iota(jnp.int32, (SEG, T, 1), 1)
    w = jnp.exp(-_DECAY * ((T - 1) - t_idx).astype(jnp.float32))
    w = jnp.where(t_idx.astype(jnp.float32) < lens_f, w, 0.0)
    e = jnp.sum(h_ref[...] * w, axis=1)
    lens2 = lens_f[:, :, 0]
    r = jnp.exp(jnp.float32(_DECAY))
    wsum = jnp.exp(-_DECAY * (T - 1)) * (jnp.exp(_DECAY * lens2) - 1.0) / (r - 1.0)
    wsum = jnp.maximum(wsum, _EPS)
    e = e / wsum
    mu = jnp.mean(e, axis=1, keepdims=True)
    var = jnp.mean((e - mu) ** 2, axis=1, keepdims=True)
    e_ln = (e - mu) * jax.lax.rsqrt(var + _LN_EPS) * scale_ref[...] + bias_ref[...]
    out_ref[...] = jnp.where(lens2 >= 1.0, e_ln, e)


def kernel(H, valid_lens, ln_scale, ln_bias):
    B, F, T, D = H.shape
    S = B * F
    SEG = 16
    grid = (S // SEG,)
    H3 = H.reshape(S, T, D)
    lens3 = valid_lens.reshape(S, 1, 1).astype(jnp.int32)
    scale2 = ln_scale.reshape(1, D)
    bias2 = ln_bias.reshape(1, D)

    out = pl.pallas_call(
        functools.partial(_tc_body, SEG=SEG, T=T, D=D),
        grid=grid,
        in_specs=[
            pl.BlockSpec((SEG, T, D), lambda i: (i, 0, 0)),
            pl.BlockSpec((SEG, 1, 1), lambda i: (i, 0, 0)),
            pl.BlockSpec((1, D), lambda i: (0, 0)),
            pl.BlockSpec((1, D), lambda i: (0, 0)),
        ],
        out_specs=pl.BlockSpec((SEG, D), lambda i: (i, 0)),
        out_shape=jax.ShapeDtypeStruct((S, D), jnp.float32),
    )(H3, lens3, scale2, bias2)
    return out.reshape(B, F, D)


# speedup vs baseline: 2.0141x; 1.2575x over previous
import functools

import jax
import jax.numpy as jnp
from jax.experimental import pallas as pl

_DECAY = 0.1
_EPS = 1e-8
_LN_EPS = 1e-5


def _body(h_ref, lens_ref, scale_ref, bias_ref, out_ref, *, F, T, D):
    lens_f = lens_ref[...].astype(jnp.float32)
    lens4 = lens_f.reshape(1, F, 1, 1)
    t_idx = jax.lax.broadcasted_iota(jnp.int32, (1, F, T, 1), 2)
    w = jnp.exp(-_DECAY * ((T - 1) - t_idx).astype(jnp.float32))
    w = jnp.where(t_idx.astype(jnp.float32) < lens4, w, 0.0)
    e = jnp.sum(h_ref[...] * w, axis=2)
    lens3 = lens4[:, :, :, 0]
    r = jnp.exp(jnp.float32(_DECAY))
    wsum = jnp.exp(-_DECAY * (T - 1)) * (jnp.exp(_DECAY * lens3) - 1.0) / (r - 1.0)
    wsum = jnp.maximum(wsum, _EPS)
    e = e / wsum
    mu = jnp.mean(e, axis=2, keepdims=True)
    var = jnp.mean((e - mu) ** 2, axis=2, keepdims=True)
    scale = scale_ref[...].reshape(1, 1, D)
    bias = bias_ref[...].reshape(1, 1, D)
    e_ln = (e - mu) * jax.lax.rsqrt(var + _LN_EPS) * scale + bias
    out_ref[...] = jnp.where(lens3 >= 1.0, e_ln, e)


def kernel(H, valid_lens, ln_scale, ln_bias):
    B, F, T, D = H.shape
    lens2 = valid_lens.astype(jnp.int32).reshape(B, 1, F)
    scale2 = ln_scale.reshape(1, D)
    bias2 = ln_bias.reshape(1, D)

    out = pl.pallas_call(
        functools.partial(_body, F=F, T=T, D=D),
        grid=(B,),
        in_specs=[
            pl.BlockSpec((1, F, T, D), lambda i: (i, 0, 0, 0)),
            pl.BlockSpec((1, 1, F), lambda i: (i, 0, 0)),
            pl.BlockSpec((1, D), lambda i: (0, 0)),
            pl.BlockSpec((1, D), lambda i: (0, 0)),
        ],
        out_specs=pl.BlockSpec((1, F, D), lambda i: (i, 0, 0)),
        out_shape=jax.ShapeDtypeStruct((B, F, D), jnp.float32),
    )(H, lens2, scale2, bias2)
    return out


# speedup vs baseline: 2.1521x; 1.0685x over previous
import functools

import jax
import jax.numpy as jnp
from jax.experimental import pallas as pl

_DECAY = 0.1
_EPS = 1e-8
_LN_EPS = 1e-5


def _body(h_ref, lens_ref, scale_ref, bias_ref, out_ref, *, BB, F, T, D):
    lens_f = lens_ref[...].astype(jnp.float32)
    lens4 = lens_f.reshape(BB, F, 1, 1)
    t_idx = jax.lax.broadcasted_iota(jnp.int32, (BB, F, T, 1), 2)
    w = jnp.exp(-_DECAY * ((T - 1) - t_idx).astype(jnp.float32))
    w = jnp.where(t_idx.astype(jnp.float32) < lens4, w, 0.0)
    e = jnp.sum(h_ref[...] * w, axis=2)
    lens3 = lens4[:, :, :, 0]
    r = jnp.exp(jnp.float32(_DECAY))
    wsum = jnp.exp(-_DECAY * (T - 1)) * (jnp.exp(_DECAY * lens3) - 1.0) / (r - 1.0)
    wsum = jnp.maximum(wsum, _EPS)
    e = e / wsum
    mu = jnp.mean(e, axis=2, keepdims=True)
    var = jnp.mean((e - mu) ** 2, axis=2, keepdims=True)
    scale = scale_ref[...].reshape(1, 1, D)
    bias = bias_ref[...].reshape(1, 1, D)
    e_ln = (e - mu) * jax.lax.rsqrt(var + _LN_EPS) * scale + bias
    out_ref[...] = jnp.where(lens3 >= 1.0, e_ln, e)


def kernel(H, valid_lens, ln_scale, ln_bias):
    B, F, T, D = H.shape
    BB = 4
    lens2 = valid_lens.astype(jnp.int32).reshape(B, 1, F)
    scale2 = ln_scale.reshape(1, D)
    bias2 = ln_bias.reshape(1, D)

    out = pl.pallas_call(
        functools.partial(_body, BB=BB, F=F, T=T, D=D),
        grid=(B // BB,),
        in_specs=[
            pl.BlockSpec((BB, F, T, D), lambda i: (i, 0, 0, 0)),
            pl.BlockSpec((BB, 1, F), lambda i: (i, 0, 0)),
            pl.BlockSpec((1, D), lambda i: (0, 0)),
            pl.BlockSpec((1, D), lambda i: (0, 0)),
        ],
        out_specs=pl.BlockSpec((BB, F, D), lambda i: (i, 0, 0)),
        out_shape=jax.ShapeDtypeStruct((B, F, D), jnp.float32),
    )(H, lens2, scale2, bias2)
    return out
